# baseline (speedup 1.0000x reference)
import jax
import jax.numpy as jnp
from jax.experimental import pallas as pl
from jax.experimental.pallas import tpu as pltpu

N = 2708
D = 1433
NOISE = 5
HID = 128
OUT = 128
F32 = jnp.float32

BM = 256
NBLK = 11
DBLK = 6
HBN = 512
HJ = 3
GAT_BM = 256
GAT_BN = 1408
GAT_NJ = 2


def _rng_consts():
    rk = jax.random.key(7)
    r = jax.random.split(rk, 4)
    node_noise = jax.random.bernoulli(r[0], 0.5, (N, 2, NOISE)).astype(F32)
    attr_noise = jax.random.bernoulli(r[1], 0.5, (D, 2, NOISE)).astype(F32)
    node_eps0 = jax.random.normal(r[2], (N, 1, OUT), dtype=F32)[:, 0, :]
    attr_eps0 = jax.random.normal(r[3], (D, 1, 128), dtype=F32)[:, 0, :]
    return node_noise, attr_noise, node_eps0, attr_eps0


try:
    _RNG_CONSTS = tuple(jax.device_get(t) for t in _rng_consts())
except Exception:
    _RNG_CONSTS = None


def _get_rng_consts():
    if _RNG_CONSTS is not None:
        return tuple(jnp.asarray(t) for t in _RNG_CONSTS)
    return _rng_consts()


def _dot(a, b):
    return jnp.dot(a, b, preferred_element_type=F32)


def _dot0(a, b):
    return jax.lax.dot_general(a, b, (((0,), (0,)), ((), ())),
                               preferred_element_type=F32)


def _dot1(a, b):
    return jax.lax.dot_general(a, b, (((1,), (1,)), ((), ())),
                               preferred_element_type=F32)


def _k0_body(adj_ref, o_ref):
    o_ref[...] = adj_ref[...].astype(jnp.bfloat16)


def _k0(adj):
    return pl.pallas_call(
        _k0_body,
        grid=(NBLK,),
        in_specs=[pl.BlockSpec((BM, N), lambda i: (i, 0))],
        out_specs=pl.BlockSpec((BM, N), lambda i: (i, 0)),
        out_shape=jax.ShapeDtypeStruct((N, N), jnp.bfloat16),
    )(adj)


def _k1_body(xt_ref, w_ref, nn0_ref, nn1_ref, wn_ref, b1_ref, b1v_ref, o_ref):
    acc = _dot0(xt_ref[...], w_ref[...])
    xa = acc[:, :HID] + b1_ref[...]
    g1 = acc[:, HID:] + b1v_ref[...]
    h0 = xa + _dot(nn0_ref[...], wn_ref[...])
    h1 = xa + _dot(nn1_ref[...], wn_ref[...])
    o_ref[...] = jnp.concatenate([h0, h1, g1], axis=1).astype(jnp.bfloat16)


def _k1(xt, wcat, nn0, nn1, wn, b1, b1v):
    return pl.pallas_call(
        _k1_body,
        grid=(NBLK,),
        in_specs=[
            pl.BlockSpec((D, BM), lambda i: (0, i)),
            pl.BlockSpec((D, 2 * HID), lambda i: (0, 0)),
            pl.BlockSpec((BM, NOISE), lambda i: (i, 0)),
            pl.BlockSpec((BM, NOISE), lambda i: (i, 0)),
            pl.BlockSpec((NOISE, HID), lambda i: (0, 0)),
            pl.BlockSpec((1, HID), lambda i: (0, 0)),
            pl.BlockSpec((1, HID), lambda i: (0, 0)),
        ],
        out_specs=pl.BlockSpec((BM, 3 * HID), lambda i: (i, 0)),
        out_shape=jax.ShapeDtypeStruct((N, 3 * HID), jnp.bfloat16),
    )(xt, wcat, nn0, nn1, wn, b1, b1v)


def _k2_body(adj_ref, s1_ref, w2mu_ref, b2mu_ref, w2v_ref, b2v_ref, o_ref):
    t = _dot(adj_ref[...].astype(jnp.bfloat16), s1_ref[...])
    r = jnp.maximum(t, 0.0)
    o_ref[...] = jnp.concatenate([
        _dot(r[:, :HID], w2mu_ref[...]) + b2mu_ref[...],
        _dot(r[:, HID:2 * HID], w2mu_ref[...]) + b2mu_ref[...],
        _dot(r[:, 2 * HID:], w2v_ref[...]) + b2v_ref[...],
    ], axis=1).astype(jnp.bfloat16)


def _k2(adj, s1, w2mu, b2mu, w2v, b2v):
    return pl.pallas_call(
        _k2_body,
        grid=(NBLK,),
        in_specs=[
            pl.BlockSpec((BM, N), lambda i: (i, 0)),
            pl.BlockSpec((N, 3 * HID), lambda i: (0, 0)),
            pl.BlockSpec((HID, HID), lambda i: (0, 0)),
            pl.BlockSpec((1, HID), lambda i: (0, 0)),
            pl.BlockSpec((HID, HID), lambda i: (0, 0)),
            pl.BlockSpec((1, HID), lambda i: (0, 0)),
        ],
        out_specs=pl.BlockSpec((BM, 3 * HID), lambda i: (i, 0)),
        out_shape=jax.ShapeDtypeStruct((N, 3 * HID), jnp.bfloat16),
    )(adj, s1, w2mu, b2mu, w2v, b2v)


def _k3_body(adj_ref, s2_ref, eps_ref, m_ref, z_ref, sig_ref):
    m = _dot(adj_ref[...].astype(jnp.bfloat16), s2_ref[...])
    m_ref[...] = m
    sig = jnp.exp(0.5 * m[:, 2 * HID:])
    sig_ref[...] = sig
    z_ref[...] = m[:, :HID] + eps_ref[...] * sig


def _k3(adj, s2, eps0):
    return pl.pallas_call(
        _k3_body,
        grid=(NBLK,),
        in_specs=[
            pl.BlockSpec((BM, N), lambda i: (i, 0)),
            pl.BlockSpec((N, 3 * HID), lambda i: (0, 0)),
            pl.BlockSpec((BM, HID), lambda i: (i, 0)),
        ],
        out_specs=[
            pl.BlockSpec((BM, 3 * HID), lambda i: (i, 0)),
            pl.BlockSpec((BM, HID), lambda i: (i, 0)),
            pl.BlockSpec((BM, HID), lambda i: (i, 0)),
        ],
        out_shape=[
            jax.ShapeDtypeStruct((N, 3 * HID), F32),
            jax.ShapeDtypeStruct((N, HID), F32),
            jax.ShapeDtypeStruct((N, HID), F32),
        ],
    )(adj, s2, eps0)


def _k4_body(xt_ref, w_ref, an0_ref, an1_ref, wan_ref, b1_ref, b1v_ref,
             w2mu_ref, b2mu_ref, w2v_ref, b2v_ref, aeps_ref,
             m0_ref, m1_ref, lv_ref, sig_ref, za_ref):
    a = _dot(xt_ref[...], w_ref[...])
    base = a[:, :HID] + b1_ref[...]
    n0 = _dot(an0_ref[...], wan_ref[...])
    n1 = _dot(an1_ref[...], wan_ref[...])
    u0 = jnp.maximum(base + n0, 0.0)
    u1 = jnp.maximum(base + n1, 0.0)
    v = jnp.maximum(a[:, HID:] + b1v_ref[...], 0.0)
    m0 = _dot(u0, w2mu_ref[...]) + b2mu_ref[...]
    m1 = _dot(u1, w2mu_ref[...]) + b2mu_ref[...]
    lv = _dot(v, w2v_ref[...]) + b2v_ref[...]
    sig = jnp.exp(0.5 * lv)
    m0_ref[...] = m0
    m1_ref[...] = m1
    lv_ref[...] = lv
    sig_ref[...] = sig
    za_ref[...] = m0 + aeps_ref[...] * sig


def _k4(xt, wacat, an0, an1, wan, ab1, ab1v, w2mu, b2mu, w2v, b2v, aeps0):
    spec_row = pl.BlockSpec((BM, 128), lambda i: (i, 0))
    spec_n = pl.BlockSpec((BM, NOISE), lambda i: (i, 0))
    spec_w = pl.BlockSpec((128, 128), lambda i: (0, 0))
    spec_wn = pl.BlockSpec((NOISE, 128), lambda i: (0, 0))
    spec_b = pl.BlockSpec((1, 128), lambda i: (0, 0))
    return pl.pallas_call(
        _k4_body,
        grid=(DBLK,),
        in_specs=[
            pl.BlockSpec((BM, N), lambda i: (i, 0)),
            pl.BlockSpec((N, 2 * HID), lambda i: (0, 0)),
            spec_n, spec_n, spec_wn, spec_b, spec_b,
            spec_w, spec_b, spec_w, spec_b, spec_row,
        ],
        out_specs=[spec_row] * 5,
        out_shape=[jax.ShapeDtypeStruct((D, HID), F32)] * 5,
    )(xt, wacat, an0, an1, wan, ab1, ab1v, w2mu, b2mu, w2v, b2v, aeps0)


def _k5_body(zb_ref, zall_ref, o_ref):
    o_ref[...] = _dot1(zb_ref[...], zall_ref[...])


def _k5(z_u):
    return pl.pallas_call(
        _k5_body,
        grid=(NBLK,),
        in_specs=[
            pl.BlockSpec((BM, HID), lambda i: (i, 0)),
            pl.BlockSpec((N, HID), lambda i: (0, 0)),
        ],
        out_specs=pl.BlockSpec((BM, N), lambda i: (i, 0)),
        out_shape=jax.ShapeDtypeStruct((N, N), F32),
    )(z_u, z_u)


def _k67_body(zb_ref, xt_ref, za_ref, ones_ref, wt_ref, alr_ref, ec_ref,
              cf_ref, el_ref, er_ref):
    zb = zb_ref[...]
    xt = xt_ref[...]
    xz = _dot0(xt, za_ref[...])
    rs = _dot0(jnp.abs(xt), ones_ref[...])
    fine = xz / jnp.maximum(rs, 1e-12)
    cf256 = jnp.concatenate([zb, fine], axis=1)
    wlr = _dot0(wt_ref[...], alr_ref[...])
    el_ref[...] = _dot(cf256, wlr[:, 0:1])
    er_ref[...] = _dot(cf256, wlr[:, 1:2])
    cf_ref[...] = jnp.concatenate(
        [cf256, jnp.broadcast_to(ec_ref[...], (BM, 128))], axis=1
    ).astype(jnp.bfloat16)


def _k67(z_u, xt, za, ones_d, wt, alr, ec):
    return pl.pallas_call(
        _k67_body,
        grid=(NBLK,),
        in_specs=[
            pl.BlockSpec((BM, HID), lambda i: (i, 0)),
            pl.BlockSpec((D, BM), lambda i: (0, i)),
            pl.BlockSpec((D, HID), lambda i: (0, 0)),
            pl.BlockSpec((D, 128), lambda i: (0, 0)),
            pl.BlockSpec((D, 2 * HID), lambda i: (0, 0)),
            pl.BlockSpec((D, 2), lambda i: (0, 0)),
            pl.BlockSpec((1, 128), lambda i: (0, 0)),
        ],
        out_specs=[
            pl.BlockSpec((BM, 3 * HID), lambda i: (i, 0)),
            pl.BlockSpec((BM, 1), lambda i: (i, 0)),
            pl.BlockSpec((BM, 1), lambda i: (i, 0)),
        ],
        out_shape=[
            jax.ShapeDtypeStruct((N, 3 * HID), jnp.bfloat16),
            jax.ShapeDtypeStruct((N, 1), F32),
            jax.ShapeDtypeStruct((N, 1), F32),
        ],
    )(z_u, xt, za, ones_d, wt, alr, ec)


def _k8_body(er_ref, elt_ref, adj_ref, cf_ref, wt_ref, b_ref, o_ref):
    e = er_ref[...] + elt_ref[...]
    e = jnp.maximum(e, 0.2 * e)
    e = jnp.where(adj_ref[...] > 0, e, -1e9)
    m = jnp.max(e, axis=1, keepdims=True)
    p = jnp.exp(e - m).astype(jnp.bfloat16)
    acf = _dot(p, cf_ref[...])
    l = acf[:, 2 * HID:2 * HID + 1]
    alpha = (acf[:, :2 * HID] / l).astype(jnp.bfloat16)
    out = _dot1(alpha, wt_ref[...]) + b_ref[...]
    o_ref[...] = jnp.transpose(out)


def _k8(er, elt, adj, cf, wt_bf, decb):
    return pl.pallas_call(
        _k8_body,
        grid=(NBLK,),
        in_specs=[
            pl.BlockSpec((GAT_BM, 1), lambda i: (i, 0)),
            pl.BlockSpec((1, N), lambda i: (0, 0)),
            pl.BlockSpec((GAT_BM, N), lambda i: (i, 0)),
            pl.BlockSpec((N, 3 * HID), lambda i: (0, 0)),
            pl.BlockSpec((D, 2 * HID), lambda i: (0, 0)),
            pl.BlockSpec((1, D), lambda i: (0, 0)),
        ],
        out_specs=pl.BlockSpec((D, GAT_BM), lambda i: (0, i)),
        out_shape=jax.ShapeDtypeStruct((D, N), F32),
    )(er, elt, adj, cf, wt_bf, decb)


def kernel(graph, x, nmu_W1, nmu_b1, nmu_W2, nmu_b2, nvar_W1, nvar_b1,
           nvar_W2, nvar_b2, amu_W1, amu_b1, amu_W2, amu_b2, avar_W1,
           avar_b1, avar_W2, avar_b2, dec_W, dec_al, dec_ar, dec_b):
    f32 = F32
    node_noise, attr_noise, node_eps0, attr_eps0 = _get_rng_consts()
    nn0 = node_noise[:, 0, :]
    nn1 = node_noise[:, 1, :]
    an0 = attr_noise[:, 0, :]
    an1 = attr_noise[:, 1, :]

    xt = x.T
    wt = dec_W.T

    wcat = jnp.concatenate([nmu_W1[NOISE:], nvar_W1], axis=1)
    wn = nmu_W1[:NOISE]
    b1 = nmu_b1.reshape(1, HID)
    b1v = nvar_b1.reshape(1, HID)

    wacat = jnp.concatenate([amu_W1[NOISE:], avar_W1], axis=1)
    wan = amu_W1[:NOISE]

    s1 = _k1(xt, wcat, nn0, nn1, wn, b1, b1v)
    s2 = _k2(graph, s1, nmu_W2, nmu_b2.reshape(1, HID),
             nvar_W2, nvar_b2.reshape(1, HID))
    m_all, z_u, sig_n = _k3(graph, s2, node_eps0)

    links = _k5(z_u)

    am0, am1, alv, asig, z_a = _k4(
        xt, wacat, an0, an1, wan,
        amu_b1.reshape(1, 128), avar_b1.reshape(1, 128),
        amu_W2, amu_b2.reshape(1, 128), avar_W2, avar_b2.reshape(1, 128),
        attr_eps0)

    ones_d = jnp.ones((D, 128), f32)
    alr = jnp.stack([dec_al, dec_ar], axis=1)
    ec = jnp.zeros((1, 128), f32).at[0, 0].set(1.0)
    cf, el, er = _k67(z_u, xt, z_a, ones_d, wt, alr, ec)
    elt = el.T
    out_at = _k8(er, elt, graph, cf, wt.astype(jnp.bfloat16),
                 dec_b.reshape(1, D))

    node_mu0 = m_all[:, :HID]
    node_mu1 = m_all[:, HID:2 * HID]
    node_logv = m_all[:, 2 * HID:]

    merged_node_mu = jnp.stack([node_mu1, node_mu0], axis=1)[:, None, :, :]
    sig4 = sig_n[:, None, None, :]
    z4 = z_u[:, None, None, :]
    merged_node_sigma = jnp.concatenate([sig4, sig4], axis=2)
    merged_node_z = jnp.concatenate([z4, z4], axis=2)
    node_logv, z_u_leaf, out_at = jax.lax.optimization_barrier(
        (node_logv, z_u, out_at))
    node_logv_iw = node_logv[:, None, :]
    node_z_iw = z_u_leaf[:, None, :]

    merged_attr_mu = jnp.stack([am1, am0], axis=1)[:, None, :, :]
    asig4 = asig[:, None, None, :]
    za4 = z_a[:, None, None, :]
    merged_attr_sigma = jnp.concatenate([asig4, asig4], axis=2)
    merged_attr_z = jnp.concatenate([za4, za4], axis=2)
    attr_logv_iw = alv[:, None, :]
    attr_z_iw = z_a[:, None, :]

    reconstruct_node_logits = links[:, :, None]
    reconstruct_attr_logits = out_at.T[:, :, None]

    return (merged_node_mu, merged_node_sigma, merged_node_z, node_logv_iw,
            node_z_iw, merged_attr_mu, merged_attr_sigma, merged_attr_z,
            attr_logv_iw, attr_z_iw, reconstruct_node_logits,
            reconstruct_attr_logits, node_mu0, am0)


# speedup vs baseline: 1.0597x; 1.0597x over previous
import jax
import jax.numpy as jnp
from jax.experimental import pallas as pl
from jax.experimental.pallas import tpu as pltpu

N = 2708
D = 1433
NOISE = 5
HID = 128
OUT = 128
F32 = jnp.float32

BM = 512
NBLK = 6
DBLK = 3
GAT_BM = 512


def _rng_consts():
    rk = jax.random.key(7)
    r = jax.random.split(rk, 4)
    node_noise = jax.random.bernoulli(r[0], 0.5, (N, 2, NOISE)).astype(F32)
    attr_noise = jax.random.bernoulli(r[1], 0.5, (D, 2, NOISE)).astype(F32)
    node_eps0 = jax.random.normal(r[2], (N, 1, OUT), dtype=F32)[:, 0, :]
    attr_eps0 = jax.random.normal(r[3], (D, 1, 128), dtype=F32)[:, 0, :]
    return node_noise, attr_noise, node_eps0, attr_eps0


try:
    _RNG_CONSTS = tuple(jax.device_get(t) for t in _rng_consts())
except Exception:
    _RNG_CONSTS = None


def _get_rng_consts():
    if _RNG_CONSTS is not None:
        return tuple(jnp.asarray(t) for t in _RNG_CONSTS)
    return _rng_consts()


def _dot(a, b):
    return jnp.dot(a, b, preferred_element_type=F32)


def _dot0(a, b):
    return jax.lax.dot_general(a, b, (((0,), (0,)), ((), ())),
                               preferred_element_type=F32)


def _dot1(a, b):
    return jax.lax.dot_general(a, b, (((1,), (1,)), ((), ())),
                               preferred_element_type=F32)


def _k1_body(xt_ref, w_ref, nn0_ref, nn1_ref, wn_ref, b1_ref, b1v_ref, o_ref):
    acc = _dot0(xt_ref[...], w_ref[...])
    xa = acc[:, :HID] + b1_ref[...]
    g1 = acc[:, HID:] + b1v_ref[...]
    h0 = xa + _dot(nn0_ref[...], wn_ref[...])
    h1 = xa + _dot(nn1_ref[...], wn_ref[...])
    o_ref[...] = jnp.concatenate([h0, h1, g1], axis=1).astype(jnp.bfloat16)


def _k1(xt, wcat, nn0, nn1, wn, b1, b1v):
    return pl.pallas_call(
        _k1_body,
        grid=(NBLK,),
        in_specs=[
            pl.BlockSpec((D, BM), lambda i: (0, i)),
            pl.BlockSpec((D, 2 * HID), lambda i: (0, 0)),
            pl.BlockSpec((BM, NOISE), lambda i: (i, 0)),
            pl.BlockSpec((BM, NOISE), lambda i: (i, 0)),
            pl.BlockSpec((NOISE, HID), lambda i: (0, 0)),
            pl.BlockSpec((1, HID), lambda i: (0, 0)),
            pl.BlockSpec((1, HID), lambda i: (0, 0)),
        ],
        out_specs=pl.BlockSpec((BM, 3 * HID), lambda i: (i, 0)),
        out_shape=jax.ShapeDtypeStruct((N, 3 * HID), jnp.bfloat16),
    )(xt, wcat, nn0, nn1, wn, b1, b1v)


def _k2_body(adj_ref, s1_ref, w2mu_ref, b2mu_ref, w2v_ref, b2v_ref, o_ref):
    t = _dot(adj_ref[...].astype(jnp.bfloat16), s1_ref[...])
    r = jnp.maximum(t, 0.0)
    o_ref[...] = jnp.concatenate([
        _dot(r[:, :HID], w2mu_ref[...]) + b2mu_ref[...],
        _dot(r[:, HID:2 * HID], w2mu_ref[...]) + b2mu_ref[...],
        _dot(r[:, 2 * HID:], w2v_ref[...]) + b2v_ref[...],
    ], axis=1).astype(jnp.bfloat16)


def _k2(adj, s1, w2mu, b2mu, w2v, b2v):
    return pl.pallas_call(
        _k2_body,
        grid=(NBLK,),
        in_specs=[
            pl.BlockSpec((BM, N), lambda i: (i, 0)),
            pl.BlockSpec((N, 3 * HID), lambda i: (0, 0)),
            pl.BlockSpec((HID, HID), lambda i: (0, 0)),
            pl.BlockSpec((1, HID), lambda i: (0, 0)),
            pl.BlockSpec((HID, HID), lambda i: (0, 0)),
            pl.BlockSpec((1, HID), lambda i: (0, 0)),
        ],
        out_specs=pl.BlockSpec((BM, 3 * HID), lambda i: (i, 0)),
        out_shape=jax.ShapeDtypeStruct((N, 3 * HID), jnp.bfloat16),
    )(adj, s1, w2mu, b2mu, w2v, b2v)


def _k3_body(adj_ref, s2_ref, eps_ref, m_ref, z_ref, sig_ref):
    m = _dot(adj_ref[...].astype(jnp.bfloat16), s2_ref[...])
    m_ref[...] = m
    sig = jnp.exp(0.5 * m[:, 2 * HID:])
    sig_ref[...] = sig
    z_ref[...] = m[:, :HID] + eps_ref[...] * sig


def _k3(adj, s2, eps0):
    return pl.pallas_call(
        _k3_body,
        grid=(NBLK,),
        in_specs=[
            pl.BlockSpec((BM, N), lambda i: (i, 0)),
            pl.BlockSpec((N, 3 * HID), lambda i: (0, 0)),
            pl.BlockSpec((BM, HID), lambda i: (i, 0)),
        ],
        out_specs=[
            pl.BlockSpec((BM, 3 * HID), lambda i: (i, 0)),
            pl.BlockSpec((BM, HID), lambda i: (i, 0)),
            pl.BlockSpec((BM, HID), lambda i: (i, 0)),
        ],
        out_shape=[
            jax.ShapeDtypeStruct((N, 3 * HID), F32),
            jax.ShapeDtypeStruct((N, HID), F32),
            jax.ShapeDtypeStruct((N, HID), F32),
        ],
    )(adj, s2, eps0)


def _k4_body(xt_ref, w_ref, an0_ref, an1_ref, wan_ref, b1_ref, b1v_ref,
             w2mu_ref, b2mu_ref, w2v_ref, b2v_ref, aeps_ref,
             m0_ref, m1_ref, lv_ref, sig_ref, za_ref):
    a = _dot(xt_ref[...], w_ref[...])
    base = a[:, :HID] + b1_ref[...]
    n0 = _dot(an0_ref[...], wan_ref[...])
    n1 = _dot(an1_ref[...], wan_ref[...])
    u0 = jnp.maximum(base + n0, 0.0)
    u1 = jnp.maximum(base + n1, 0.0)
    v = jnp.maximum(a[:, HID:] + b1v_ref[...], 0.0)
    m0 = _dot(u0, w2mu_ref[...]) + b2mu_ref[...]
    m1 = _dot(u1, w2mu_ref[...]) + b2mu_ref[...]
    lv = _dot(v, w2v_ref[...]) + b2v_ref[...]
    sig = jnp.exp(0.5 * lv)
    m0_ref[...] = m0
    m1_ref[...] = m1
    lv_ref[...] = lv
    sig_ref[...] = sig
    za_ref[...] = m0 + aeps_ref[...] * sig


def _k4(xt, wacat, an0, an1, wan, ab1, ab1v, w2mu, b2mu, w2v, b2v, aeps0):
    spec_row = pl.BlockSpec((BM, 128), lambda i: (i, 0))
    spec_n = pl.BlockSpec((BM, NOISE), lambda i: (i, 0))
    spec_w = pl.BlockSpec((128, 128), lambda i: (0, 0))
    spec_wn = pl.BlockSpec((NOISE, 128), lambda i: (0, 0))
    spec_b = pl.BlockSpec((1, 128), lambda i: (0, 0))
    return pl.pallas_call(
        _k4_body,
        grid=(DBLK,),
        in_specs=[
            pl.BlockSpec((BM, N), lambda i: (i, 0)),
            pl.BlockSpec((N, 2 * HID), lambda i: (0, 0)),
            spec_n, spec_n, spec_wn, spec_b, spec_b,
            spec_w, spec_b, spec_w, spec_b, spec_row,
        ],
        out_specs=[spec_row] * 5,
        out_shape=[jax.ShapeDtypeStruct((D, HID), F32)] * 5,
    )(xt, wacat, an0, an1, wan, ab1, ab1v, w2mu, b2mu, w2v, b2v, aeps0)


def _k5_body(zb_ref, zall_ref, o_ref):
    o_ref[...] = _dot1(zb_ref[...], zall_ref[...])


def _k5(z_u):
    return pl.pallas_call(
        _k5_body,
        grid=(NBLK,),
        in_specs=[
            pl.BlockSpec((BM, HID), lambda i: (i, 0)),
            pl.BlockSpec((N, HID), lambda i: (0, 0)),
        ],
        out_specs=pl.BlockSpec((BM, N), lambda i: (i, 0)),
        out_shape=jax.ShapeDtypeStruct((N, N), F32),
    )(z_u, z_u)


def _k67_body(zb_ref, xt_ref, za_ref, ones_ref, wt_ref, alr_ref, ec_ref,
              cf_ref, el_ref, er_ref):
    zb = zb_ref[...]
    xt = xt_ref[...]
    xz = _dot0(xt, za_ref[...])
    rs = _dot0(jnp.abs(xt), ones_ref[...])
    fine = xz / jnp.maximum(rs, 1e-12)
    cf256 = jnp.concatenate([zb, fine], axis=1)
    wlr = _dot0(wt_ref[...], alr_ref[...])
    el_ref[...] = _dot(cf256, wlr[:, 0:1])
    er_ref[...] = _dot(cf256, wlr[:, 1:2])
    cf_ref[...] = jnp.concatenate(
        [cf256, jnp.broadcast_to(ec_ref[...], (BM, 128))], axis=1
    ).astype(jnp.bfloat16)


def _k67(z_u, xt, za, ones_d, wt, alr, ec):
    return pl.pallas_call(
        _k67_body,
        grid=(NBLK,),
        in_specs=[
            pl.BlockSpec((BM, HID), lambda i: (i, 0)),
            pl.BlockSpec((D, BM), lambda i: (0, i)),
            pl.BlockSpec((D, HID), lambda i: (0, 0)),
            pl.BlockSpec((D, 128), lambda i: (0, 0)),
            pl.BlockSpec((D, 2 * HID), lambda i: (0, 0)),
            pl.BlockSpec((D, 2), lambda i: (0, 0)),
            pl.BlockSpec((1, 128), lambda i: (0, 0)),
        ],
        out_specs=[
            pl.BlockSpec((BM, 3 * HID), lambda i: (i, 0)),
            pl.BlockSpec((BM, 1), lambda i: (i, 0)),
            pl.BlockSpec((BM, 1), lambda i: (i, 0)),
        ],
        out_shape=[
            jax.ShapeDtypeStruct((N, 3 * HID), jnp.bfloat16),
            jax.ShapeDtypeStruct((N, 1), F32),
            jax.ShapeDtypeStruct((N, 1), F32),
        ],
    )(z_u, xt, za, ones_d, wt, alr, ec)


def _k8_body(er_ref, elt_ref, adj_ref, cf_ref, wt_ref, b_ref, o_ref):
    e = er_ref[...] + elt_ref[...]
    e = jnp.maximum(e, 0.2 * e)
    e = jnp.where(adj_ref[...] > 0, e, -1e9)
    m = jnp.max(e, axis=1, keepdims=True)
    p = jnp.exp(e - m).astype(jnp.bfloat16)
    acf = _dot(p, cf_ref[...])
    l = acf[:, 2 * HID:2 * HID + 1]
    alpha = (acf[:, :2 * HID] / l).astype(jnp.bfloat16)
    out = _dot1(alpha, wt_ref[...]) + b_ref[...]
    o_ref[...] = jnp.transpose(out)


def _k8(er, elt, adj, cf, wt_bf, decb):
    return pl.pallas_call(
        _k8_body,
        grid=(NBLK,),
        in_specs=[
            pl.BlockSpec((GAT_BM, 1), lambda i: (i, 0)),
            pl.BlockSpec((1, N), lambda i: (0, 0)),
            pl.BlockSpec((GAT_BM, N), lambda i: (i, 0)),
            pl.BlockSpec((N, 3 * HID), lambda i: (0, 0)),
            pl.BlockSpec((D, 2 * HID), lambda i: (0, 0)),
            pl.BlockSpec((1, D), lambda i: (0, 0)),
        ],
        out_specs=pl.BlockSpec((D, GAT_BM), lambda i: (0, i)),
        out_shape=jax.ShapeDtypeStruct((D, N), F32),
    )(er, elt, adj, cf, wt_bf, decb)


def kernel(graph, x, nmu_W1, nmu_b1, nmu_W2, nmu_b2, nvar_W1, nvar_b1,
           nvar_W2, nvar_b2, amu_W1, amu_b1, amu_W2, amu_b2, avar_W1,
           avar_b1, avar_W2, avar_b2, dec_W, dec_al, dec_ar, dec_b):
    f32 = F32
    node_noise, attr_noise, node_eps0, attr_eps0 = _get_rng_consts()
    nn0 = node_noise[:, 0, :]
    nn1 = node_noise[:, 1, :]
    an0 = attr_noise[:, 0, :]
    an1 = attr_noise[:, 1, :]

    xt = x.T
    wt = dec_W.T

    wcat = jnp.concatenate([nmu_W1[NOISE:], nvar_W1], axis=1)
    wn = nmu_W1[:NOISE]
    b1 = nmu_b1.reshape(1, HID)
    b1v = nvar_b1.reshape(1, HID)

    wacat = jnp.concatenate([amu_W1[NOISE:], avar_W1], axis=1)
    wan = amu_W1[:NOISE]

    s1 = _k1(xt, wcat, nn0, nn1, wn, b1, b1v)
    s2 = _k2(graph, s1, nmu_W2, nmu_b2.reshape(1, HID),
             nvar_W2, nvar_b2.reshape(1, HID))
    m_all, z_u, sig_n = _k3(graph, s2, node_eps0)

    links = _k5(z_u)

    am0, am1, alv, asig, z_a = _k4(
        xt, wacat, an0, an1, wan,
        amu_b1.reshape(1, 128), avar_b1.reshape(1, 128),
        amu_W2, amu_b2.reshape(1, 128), avar_W2, avar_b2.reshape(1, 128),
        attr_eps0)

    ones_d = jnp.ones((D, 128), f32)
    alr = jnp.stack([dec_al, dec_ar], axis=1)
    ec = jnp.zeros((1, 128), f32).at[0, 0].set(1.0)
    cf, el, er = _k67(z_u, xt, z_a, ones_d, wt, alr, ec)
    elt = el.T
    out_at = _k8(er, elt, graph, cf, wt.astype(jnp.bfloat16),
                 dec_b.reshape(1, D))

    node_mu0 = m_all[:, :HID]
    node_mu1 = m_all[:, HID:2 * HID]
    node_logv = m_all[:, 2 * HID:]

    merged_node_mu = jnp.stack([node_mu1, node_mu0], axis=1)[:, None, :, :]
    sig4 = sig_n[:, None, None, :]
    z4 = z_u[:, None, None, :]
    merged_node_sigma = jnp.concatenate([sig4, sig4], axis=2)
    merged_node_z = jnp.concatenate([z4, z4], axis=2)
    node_logv_iw = node_logv[:, None, :]
    node_z_iw = z_u[:, None, :]

    merged_attr_mu = jnp.stack([am1, am0], axis=1)[:, None, :, :]
    asig4 = asig[:, None, None, :]
    za4 = z_a[:, None, None, :]
    merged_attr_sigma = jnp.concatenate([asig4, asig4], axis=2)
    merged_attr_z = jnp.concatenate([za4, za4], axis=2)
    attr_logv_iw = alv[:, None, :]
    attr_z_iw = z_a[:, None, :]

    reconstruct_node_logits = links[:, :, None]
    reconstruct_attr_logits = out_at.T[:, :, None]

    return (merged_node_mu, merged_node_sigma, merged_node_z, node_logv_iw,
            node_z_iw, merged_attr_mu, merged_attr_sigma, merged_attr_z,
            attr_logv_iw, attr_z_iw, reconstruct_node_logits,
            reconstruct_attr_logits, node_mu0, am0)


# speedup vs baseline: 1.0697x; 1.0094x over previous
import jax
import jax.numpy as jnp
from jax.experimental import pallas as pl
from jax.experimental.pallas import tpu as pltpu

N = 2708
D = 1433
NOISE = 5
HID = 128
OUT = 128
F32 = jnp.float32

BM = 768
NBLK = 4
DBLK = 2
GAT_BM = 768


def _rng_consts():
    rk = jax.random.key(7)
    r = jax.random.split(rk, 4)
    node_noise = jax.random.bernoulli(r[0], 0.5, (N, 2, NOISE)).astype(F32)
    attr_noise = jax.random.bernoulli(r[1], 0.5, (D, 2, NOISE)).astype(F32)
    node_eps0 = jax.random.normal(r[2], (N, 1, OUT), dtype=F32)[:, 0, :]
    attr_eps0 = jax.random.normal(r[3], (D, 1, 128), dtype=F32)[:, 0, :]
    return node_noise, attr_noise, node_eps0, attr_eps0


try:
    _RNG_CONSTS = tuple(jax.device_get(t) for t in _rng_consts())
except Exception:
    _RNG_CONSTS = None


def _get_rng_consts():
    if _RNG_CONSTS is not None:
        return tuple(jnp.asarray(t) for t in _RNG_CONSTS)
    return _rng_consts()


def _dot(a, b):
    return jnp.dot(a, b, preferred_element_type=F32)


def _dot0(a, b):
    return jax.lax.dot_general(a, b, (((0,), (0,)), ((), ())),
                               preferred_element_type=F32)


def _dot1(a, b):
    return jax.lax.dot_general(a, b, (((1,), (1,)), ((), ())),
                               preferred_element_type=F32)


def _k1_body(xt_ref, w_ref, nn0_ref, nn1_ref, wn_ref, b1_ref, b1v_ref, o_ref):
    acc = _dot0(xt_ref[...], w_ref[...])
    xa = acc[:, :HID] + b1_ref[...]
    g1 = acc[:, HID:] + b1v_ref[...]
    h0 = xa + _dot(nn0_ref[...], wn_ref[...])
    h1 = xa + _dot(nn1_ref[...], wn_ref[...])
    o_ref[...] = jnp.concatenate([h0, h1, g1], axis=1).astype(jnp.bfloat16)


def _k1(xt, wcat, nn0, nn1, wn, b1, b1v):
    return pl.pallas_call(
        _k1_body,
        grid=(NBLK,),
        in_specs=[
            pl.BlockSpec((D, BM), lambda i: (0, i)),
            pl.BlockSpec((D, 2 * HID), lambda i: (0, 0)),
            pl.BlockSpec((BM, NOISE), lambda i: (i, 0)),
            pl.BlockSpec((BM, NOISE), lambda i: (i, 0)),
            pl.BlockSpec((NOISE, HID), lambda i: (0, 0)),
            pl.BlockSpec((1, HID), lambda i: (0, 0)),
            pl.BlockSpec((1, HID), lambda i: (0, 0)),
        ],
        out_specs=pl.BlockSpec((BM, 3 * HID), lambda i: (i, 0)),
        out_shape=jax.ShapeDtypeStruct((N, 3 * HID), jnp.bfloat16),
    )(xt, wcat, nn0, nn1, wn, b1, b1v)


def _k2_body(adj_ref, s1_ref, w2mu_ref, b2mu_ref, w2v_ref, b2v_ref, o_ref):
    t = _dot(adj_ref[...].astype(jnp.bfloat16), s1_ref[...])
    r = jnp.maximum(t, 0.0)
    o_ref[...] = jnp.concatenate([
        _dot(r[:, :HID], w2mu_ref[...]) + b2mu_ref[...],
        _dot(r[:, HID:2 * HID], w2mu_ref[...]) + b2mu_ref[...],
        _dot(r[:, 2 * HID:], w2v_ref[...]) + b2v_ref[...],
    ], axis=1).astype(jnp.bfloat16)


def _k2(adj, s1, w2mu, b2mu, w2v, b2v):
    return pl.pallas_call(
        _k2_body,
        grid=(NBLK,),
        in_specs=[
            pl.BlockSpec((BM, N), lambda i: (i, 0)),
            pl.BlockSpec((N, 3 * HID), lambda i: (0, 0)),
            pl.BlockSpec((HID, HID), lambda i: (0, 0)),
            pl.BlockSpec((1, HID), lambda i: (0, 0)),
            pl.BlockSpec((HID, HID), lambda i: (0, 0)),
            pl.BlockSpec((1, HID), lambda i: (0, 0)),
        ],
        out_specs=pl.BlockSpec((BM, 3 * HID), lambda i: (i, 0)),
        out_shape=jax.ShapeDtypeStruct((N, 3 * HID), jnp.bfloat16),
    )(adj, s1, w2mu, b2mu, w2v, b2v)


def _k3_body(adj_ref, s2_ref, eps_ref, m_ref, z_ref, sig_ref):
    m = _dot(adj_ref[...].astype(jnp.bfloat16), s2_ref[...])
    m_ref[...] = m
    sig = jnp.exp(0.5 * m[:, 2 * HID:])
    sig_ref[...] = sig
    z_ref[...] = m[:, :HID] + eps_ref[...] * sig


def _k3(adj, s2, eps0):
    return pl.pallas_call(
        _k3_body,
        grid=(NBLK,),
        in_specs=[
            pl.BlockSpec((BM, N), lambda i: (i, 0)),
            pl.BlockSpec((N, 3 * HID), lambda i: (0, 0)),
            pl.BlockSpec((BM, HID), lambda i: (i, 0)),
        ],
        out_specs=[
            pl.BlockSpec((BM, 3 * HID), lambda i: (i, 0)),
            pl.BlockSpec((BM, HID), lambda i: (i, 0)),
            pl.BlockSpec((BM, HID), lambda i: (i, 0)),
        ],
        out_shape=[
            jax.ShapeDtypeStruct((N, 3 * HID), F32),
            jax.ShapeDtypeStruct((N, HID), F32),
            jax.ShapeDtypeStruct((N, HID), F32),
        ],
    )(adj, s2, eps0)


def _k4_body(xt_ref, w_ref, an0_ref, an1_ref, wan_ref, b1_ref, b1v_ref,
             w2mu_ref, b2mu_ref, w2v_ref, b2v_ref, aeps_ref,
             m0_ref, m1_ref, lv_ref, sig_ref, za_ref):
    a = _dot(xt_ref[...], w_ref[...])
    base = a[:, :HID] + b1_ref[...]
    n0 = _dot(an0_ref[...], wan_ref[...])
    n1 = _dot(an1_ref[...], wan_ref[...])
    u0 = jnp.maximum(base + n0, 0.0)
    u1 = jnp.maximum(base + n1, 0.0)
    v = jnp.maximum(a[:, HID:] + b1v_ref[...], 0.0)
    m0 = _dot(u0, w2mu_ref[...]) + b2mu_ref[...]
    m1 = _dot(u1, w2mu_ref[...]) + b2mu_ref[...]
    lv = _dot(v, w2v_ref[...]) + b2v_ref[...]
    sig = jnp.exp(0.5 * lv)
    m0_ref[...] = m0
    m1_ref[...] = m1
    lv_ref[...] = lv
    sig_ref[...] = sig
    za_ref[...] = m0 + aeps_ref[...] * sig


def _k4(xt, wacat, an0, an1, wan, ab1, ab1v, w2mu, b2mu, w2v, b2v, aeps0):
    spec_row = pl.BlockSpec((BM, 128), lambda i: (i, 0))
    spec_n = pl.BlockSpec((BM, NOISE), lambda i: (i, 0))
    spec_w = pl.BlockSpec((128, 128), lambda i: (0, 0))
    spec_wn = pl.BlockSpec((NOISE, 128), lambda i: (0, 0))
    spec_b = pl.BlockSpec((1, 128), lambda i: (0, 0))
    return pl.pallas_call(
        _k4_body,
        grid=(DBLK,),
        in_specs=[
            pl.BlockSpec((BM, N), lambda i: (i, 0)),
            pl.BlockSpec((N, 2 * HID), lambda i: (0, 0)),
            spec_n, spec_n, spec_wn, spec_b, spec_b,
            spec_w, spec_b, spec_w, spec_b, spec_row,
        ],
        out_specs=[spec_row] * 5,
        out_shape=[jax.ShapeDtypeStruct((D, HID), F32)] * 5,
    )(xt, wacat, an0, an1, wan, ab1, ab1v, w2mu, b2mu, w2v, b2v, aeps0)


def _k5_body(zb_ref, zall_ref, o_ref):
    o_ref[...] = _dot1(zb_ref[...], zall_ref[...])


def _k5(z_u):
    return pl.pallas_call(
        _k5_body,
        grid=(NBLK,),
        in_specs=[
            pl.BlockSpec((BM, HID), lambda i: (i, 0)),
            pl.BlockSpec((N, HID), lambda i: (0, 0)),
        ],
        out_specs=pl.BlockSpec((BM, N), lambda i: (i, 0)),
        out_shape=jax.ShapeDtypeStruct((N, N), F32),
    )(z_u, z_u)


def _k67_body(zb_ref, xt_ref, za_ref, ones_ref, wt_ref, alr_ref, ec_ref,
              cf_ref, el_ref, er_ref):
    zb = zb_ref[...]
    xt = xt_ref[...]
    xz = _dot0(xt, za_ref[...])
    rs = _dot0(jnp.abs(xt), ones_ref[...])
    fine = xz / jnp.maximum(rs, 1e-12)
    cf256 = jnp.concatenate([zb, fine], axis=1)
    wlr = _dot0(wt_ref[...], alr_ref[...])
    el_ref[...] = _dot(cf256, wlr[:, 0:1])
    er_ref[...] = _dot(cf256, wlr[:, 1:2])
    cf_ref[...] = jnp.concatenate(
        [cf256, jnp.broadcast_to(ec_ref[...], (BM, 128))], axis=1
    ).astype(jnp.bfloat16)


def _k67(z_u, xt, za, ones_d, wt, alr, ec):
    return pl.pallas_call(
        _k67_body,
        grid=(NBLK,),
        in_specs=[
            pl.BlockSpec((BM, HID), lambda i: (i, 0)),
            pl.BlockSpec((D, BM), lambda i: (0, i)),
            pl.BlockSpec((D, HID), lambda i: (0, 0)),
            pl.BlockSpec((D, 128), lambda i: (0, 0)),
            pl.BlockSpec((D, 2 * HID), lambda i: (0, 0)),
            pl.BlockSpec((D, 2), lambda i: (0, 0)),
            pl.BlockSpec((1, 128), lambda i: (0, 0)),
        ],
        out_specs=[
            pl.BlockSpec((BM, 3 * HID), lambda i: (i, 0)),
            pl.BlockSpec((BM, 1), lambda i: (i, 0)),
            pl.BlockSpec((BM, 1), lambda i: (i, 0)),
        ],
        out_shape=[
            jax.ShapeDtypeStruct((N, 3 * HID), jnp.bfloat16),
            jax.ShapeDtypeStruct((N, 1), F32),
            jax.ShapeDtypeStruct((N, 1), F32),
        ],
    )(z_u, xt, za, ones_d, wt, alr, ec)


def _k8_body(er_ref, elt_ref, adj_ref, cf_ref, wt_ref, b_ref, o_ref):
    e = er_ref[...] + elt_ref[...]
    e = jnp.maximum(e, 0.2 * e)
    e = jnp.where(adj_ref[...] > 0, e, -1e9)
    m = jnp.max(e, axis=1, keepdims=True)
    p = jnp.exp(e - m).astype(jnp.bfloat16)
    acf = _dot(p, cf_ref[...])
    l = acf[:, 2 * HID:2 * HID + 1]
    alpha = (acf[:, :2 * HID] / l).astype(jnp.bfloat16)
    out = _dot1(alpha, wt_ref[...]) + b_ref[...]
    o_ref[...] = jnp.transpose(out)


def _k8(er, elt, adj, cf, wt_bf, decb):
    return pl.pallas_call(
        _k8_body,
        grid=(NBLK,),
        in_specs=[
            pl.BlockSpec((GAT_BM, 1), lambda i: (i, 0)),
            pl.BlockSpec((1, N), lambda i: (0, 0)),
            pl.BlockSpec((GAT_BM, N), lambda i: (i, 0)),
            pl.BlockSpec((N, 3 * HID), lambda i: (0, 0)),
            pl.BlockSpec((D, 2 * HID), lambda i: (0, 0)),
            pl.BlockSpec((1, D), lambda i: (0, 0)),
        ],
        out_specs=pl.BlockSpec((D, GAT_BM), lambda i: (0, i)),
        out_shape=jax.ShapeDtypeStruct((D, N), F32),
    )(er, elt, adj, cf, wt_bf, decb)


def kernel(graph, x, nmu_W1, nmu_b1, nmu_W2, nmu_b2, nvar_W1, nvar_b1,
           nvar_W2, nvar_b2, amu_W1, amu_b1, amu_W2, amu_b2, avar_W1,
           avar_b1, avar_W2, avar_b2, dec_W, dec_al, dec_ar, dec_b):
    f32 = F32
    node_noise, attr_noise, node_eps0, attr_eps0 = _get_rng_consts()
    nn0 = node_noise[:, 0, :]
    nn1 = node_noise[:, 1, :]
    an0 = attr_noise[:, 0, :]
    an1 = attr_noise[:, 1, :]

    xt = x.T
    wt = dec_W.T

    wcat = jnp.concatenate([nmu_W1[NOISE:], nvar_W1], axis=1)
    wn = nmu_W1[:NOISE]
    b1 = nmu_b1.reshape(1, HID)
    b1v = nvar_b1.reshape(1, HID)

    wacat = jnp.concatenate([amu_W1[NOISE:], avar_W1], axis=1)
    wan = amu_W1[:NOISE]

    s1 = _k1(xt, wcat, nn0, nn1, wn, b1, b1v)
    s2 = _k2(graph, s1, nmu_W2, nmu_b2.reshape(1, HID),
             nvar_W2, nvar_b2.reshape(1, HID))
    m_all, z_u, sig_n = _k3(graph, s2, node_eps0)

    links = _k5(z_u)

    am0, am1, alv, asig, z_a = _k4(
        xt, wacat, an0, an1, wan,
        amu_b1.reshape(1, 128), avar_b1.reshape(1, 128),
        amu_W2, amu_b2.reshape(1, 128), avar_W2, avar_b2.reshape(1, 128),
        attr_eps0)

    ones_d = jnp.ones((D, 128), f32)
    alr = jnp.stack([dec_al, dec_ar], axis=1)
    ec = jnp.zeros((1, 128), f32).at[0, 0].set(1.0)
    cf, el, er = _k67(z_u, xt, z_a, ones_d, wt, alr, ec)
    elt = el.T
    out_at = _k8(er, elt, graph, cf, wt.astype(jnp.bfloat16),
                 dec_b.reshape(1, D))

    node_mu0 = m_all[:, :HID]
    node_mu1 = m_all[:, HID:2 * HID]
    node_logv = m_all[:, 2 * HID:]

    merged_node_mu = jnp.stack([node_mu1, node_mu0], axis=1)[:, None, :, :]
    sig4 = sig_n[:, None, None, :]
    z4 = z_u[:, None, None, :]
    merged_node_sigma = jnp.concatenate([sig4, sig4], axis=2)
    merged_node_z = jnp.concatenate([z4, z4], axis=2)
    node_logv_iw = node_logv[:, None, :]
    node_z_iw = z_u[:, None, :]

    merged_attr_mu = jnp.stack([am1, am0], axis=1)[:, None, :, :]
    asig4 = asig[:, None, None, :]
    za4 = z_a[:, None, None, :]
    merged_attr_sigma = jnp.concatenate([asig4, asig4], axis=2)
    merged_attr_z = jnp.concatenate([za4, za4], axis=2)
    attr_logv_iw = alv[:, None, :]
    attr_z_iw = z_a[:, None, :]

    reconstruct_node_logits = links[:, :, None]
    reconstruct_attr_logits = out_at.T[:, :, None]

    return (merged_node_mu, merged_node_sigma, merged_node_z, node_logv_iw,
            node_z_iw, merged_attr_mu, merged_attr_sigma, merged_attr_z,
            attr_logv_iw, attr_z_iw, reconstruct_node_logits,
            reconstruct_attr_logits, node_mu0, am0)
